# baseline (speedup 1.0000x reference)
import functools

import jax
import jax.numpy as jnp
from jax import lax
from jax.experimental import pallas as pl
from jax.experimental.pallas import tpu as pltpu
from jax.experimental.pallas import tpu_sc as plsc

_VOCAB = 1000000
_DIM = 64
_B = 4096
_S = 200

_NC = 2
_NS = 16
_NW = _NC * _NS

_UNITS = _VOCAB // 128
_TAIL = _VOCAB - _UNITS * 128
_UNITS_LO = _UNITS // _NW
_UNITS_EXTRA = _UNITS - _UNITS_LO * _NW

_mesh = plsc.VectorSubcoreMesh(core_axis_name="c", subcore_axis_name="s")


def _iota16():
    return lax.broadcasted_iota(jnp.int32, (16,), 0)


@functools.partial(
    pl.kernel,
    mesh=_mesh,
    out_type=jax.ShapeDtypeStruct((_VOCAB // 2, 128), jnp.float32),
    scratch_types=[
        pltpu.VMEM((2, _DIM, 128), jnp.float32),
        pltpu.VMEM((2, _DIM, 128), jnp.float32),
        pltpu.SemaphoreType.DMA,
        pltpu.SemaphoreType.DMA,
    ],
    compiler_params=pltpu.CompilerParams(use_tc_tiling_on_sc=True, needs_layout_passes=False),
)
def _repack(tab_t_hbm, tail_hbm, packed_hbm, in_v, slab_v, ld_sem, st_sem):
    wid = lax.axis_index("s") * _NC + lax.axis_index("c")
    n_units = jnp.where(wid < _UNITS_EXTRA, _UNITS_LO + 1, _UNITS_LO)

    def unit_of(i):
        return wid + i * _NW

    def load_start(i, m):
        pltpu.async_copy(
            tab_t_hbm.at[:, pl.ds(unit_of(i) * 128, 128)], in_v.at[m], ld_sem
        )

    def load_wait(m):
        pltpu.make_async_copy(
            tab_t_hbm.at[:, pl.ds(0, 128)], in_v.at[m], ld_sem
        ).wait()

    def store_start(i, m):
        pltpu.async_copy(
            slab_v.at[m], packed_hbm.at[pl.ds(unit_of(i) * 64, 64)], st_sem
        )

    def store_wait(m):
        pltpu.make_async_copy(
            slab_v.at[m], packed_hbm.at[pl.ds(0, 64)], st_sem
        ).wait()

    def transpose_unit(m):
        src = in_v.at[m]
        dst = slab_v.at[m]
        iota = _iota16()
        two_iota = iota + iota
        perms = [lax.bitwise_and(iota + j, 15) for j in range(16)]

        def do_cb(cb, carry):
            half = lax.shift_right_logical(cb, 2)
            cbm16 = lax.shift_left(lax.bitwise_and(cb, 3), 4)
            cb16 = lax.shift_left(cb, 4)
            for pb in range(4):
                drows = iota + (16 * pb)
                sbase = two_iota + half + (32 * pb)
                for jg in range(2):
                    vals = []
                    for j in range(8 * jg, 8 * jg + 8):
                        srows = perms[j] + cbm16
                        vals.append(plsc.load_gather(src, [srows, sbase]))
                    for i, j in enumerate(range(8 * jg, 8 * jg + 8)):
                        dcols = perms[j] + cb16
                        plsc.store_scatter(dst, [drows, dcols], vals[i])
            return carry

        lax.fori_loop(0, 8, do_cb, 0)

    load_start(0, 0)

    @pl.when(n_units > 1)
    def _():
        load_start(1, 1)

    def body(i, carry):
        m = i % 2
        load_wait(m)

        @pl.when(i >= 2)
        def _():
            store_wait(m)

        transpose_unit(m)
        store_start(i, m)

        @pl.when(i + 2 < n_units)
        def _():
            load_start(i + 2, m)

        return carry

    lax.fori_loop(0, n_units, body, 0)

    @pl.when(n_units >= 2)
    def _():
        store_wait((n_units - 2) % 2)

    store_wait((n_units - 1) % 2)

    @pl.when(wid == 0)
    def _():
        pltpu.sync_copy(tail_hbm, in_v.at[0, pl.ds(0, 32)])
        pltpu.sync_copy(
            in_v.at[0, pl.ds(0, 32)],
            packed_hbm.at[pl.ds(_UNITS * 64, 32)],
        )


@functools.partial(
    pl.kernel,
    mesh=_mesh,
    out_type=jax.ShapeDtypeStruct((_S, _DIM, _B), jnp.float32),
    scratch_types=[
        pltpu.VMEM((_S, 128), jnp.int32),
        pltpu.VMEM((_S, 128), jnp.int32),
        pltpu.VMEM((2, 128, 128), jnp.float32),
        pltpu.VMEM((2, _DIM, 128), jnp.float32),
        pltpu.SemaphoreType.DMA,
        pltpu.SemaphoreType.DMA,
    ],
    compiler_params=pltpu.CompilerParams(use_tc_tiling_on_sc=True, needs_layout_passes=False),
)
def _lookup(ids_t_hbm, packed_hbm, out_t_hbm, idx_v, sel_v, buf_v, slab_v,
            gat_sem, st_sem):
    wid = lax.axis_index("s") * _NC + lax.axis_index("c")
    bcol = wid * 128

    pltpu.sync_copy(ids_t_hbm.at[:, pl.ds(bcol, 128)], idx_v)

    def prep_row(r, carry):
        for k in range(8):
            v = idx_v[r, pl.ds(16 * k, 16)]
            idx_v[r, pl.ds(16 * k, 16)] = lax.shift_right_logical(v, 1)
            sel_v[r, pl.ds(16 * k, 16)] = lax.shift_left(
                lax.bitwise_and(v, 1), 6
            )
        return carry

    lax.fori_loop(0, _S, prep_row, 0)

    def gather_start(s, m):
        pltpu.async_copy(packed_hbm.at[idx_v.at[s]], buf_v.at[m], gat_sem)

    def gather_wait(s, m):
        pltpu.make_async_copy(
            packed_hbm.at[idx_v.at[s]], buf_v.at[m], gat_sem
        ).wait()

    def store_start(s, m):
        pltpu.async_copy(
            slab_v.at[m], out_t_hbm.at[s, :, pl.ds(bcol, 128)], st_sem
        )

    def store_wait(m):
        pltpu.make_async_copy(
            slab_v.at[m], out_t_hbm.at[0, :, pl.ds(bcol, 128)], st_sem
        ).wait()

    def transpose_tokens(s, bm, sm):
        src = buf_v.at[bm]
        dst = slab_v.at[sm]
        iota = _iota16()
        srow = jnp.full((16,), 0, jnp.int32) + s
        perms = [lax.bitwise_and(iota + j, 15) for j in range(16)]

        def do_jb(jb, carry):
            toks = iota + lax.shift_left(jb, 4)
            selv = plsc.load_gather(sel_v, [srow, toks])
            for jp in range(8):
                dvecs = [perms[2 * jp + a] + (16 * db)
                         for a in range(2) for db in range(4)]
                vals = [plsc.load_gather(src, [toks, selv + dv])
                        for dv in dvecs]
                for dv, val in zip(dvecs, vals):
                    plsc.store_scatter(dst, [dv, toks], val)
            return carry

        lax.fori_loop(0, 8, do_jb, 0)

    gather_start(0, 0)
    gather_start(1, 1)

    def body(s, carry):
        bm = s % 2
        sm = s % 2
        gather_wait(s, bm)

        @pl.when(s >= 2)
        def _():
            store_wait(sm)

        transpose_tokens(s, bm, sm)
        store_start(s, sm)

        @pl.when(s + 2 < _S)
        def _():
            gather_start(s + 2, bm)

        return carry

    lax.fori_loop(0, _S, body, 0)

    store_wait(0)
    store_wait(1)


def kernel(input_ids, tok_emb):
    tail = tok_emb[_UNITS * 128:].reshape(32, 128)
    packed = _repack(tok_emb.T, tail)
    out_t = _lookup(input_ids.T, packed)
    return out_t.transpose(2, 0, 1)


# speedup vs baseline: 1.0422x; 1.0422x over previous
import functools

import jax
import jax.numpy as jnp
from jax import lax
from jax.experimental import pallas as pl
from jax.experimental.pallas import tpu as pltpu
from jax.experimental.pallas import tpu_sc as plsc

_VOCAB = 1000000
_DIM = 64
_B = 4096
_S = 200

_NC = 2
_NS = 16
_NW = _NC * _NS

_UNITS = _VOCAB // 128
_TAIL = _VOCAB - _UNITS * 128
_UNITS_LO = _UNITS // _NW
_UNITS_EXTRA = _UNITS - _UNITS_LO * _NW

_mesh = plsc.VectorSubcoreMesh(core_axis_name="c", subcore_axis_name="s")


def _iota16():
    return lax.broadcasted_iota(jnp.int32, (16,), 0)


@functools.partial(
    pl.kernel,
    mesh=_mesh,
    out_type=jax.ShapeDtypeStruct((_VOCAB // 2, 128), jnp.float32),
    scratch_types=[
        pltpu.VMEM((2, _DIM, 128), jnp.float32),
        pltpu.VMEM((2, _DIM, 128), jnp.float32),
        pltpu.SemaphoreType.DMA,
        pltpu.SemaphoreType.DMA,
    ],
    compiler_params=pltpu.CompilerParams(use_tc_tiling_on_sc=True, needs_layout_passes=False),
)
def _repack(tab_t_hbm, tail_hbm, packed_hbm, in_v, slab_v, ld_sem, st_sem):
    wid = lax.axis_index("s") * _NC + lax.axis_index("c")
    n_units = jnp.where(wid < _UNITS_EXTRA, _UNITS_LO + 1, _UNITS_LO)

    def unit_of(i):
        return wid + i * _NW

    def load_start(i, m):
        pltpu.async_copy(
            tab_t_hbm.at[:, pl.ds(unit_of(i) * 128, 128)], in_v.at[m], ld_sem
        )

    def load_wait(m):
        pltpu.make_async_copy(
            tab_t_hbm.at[:, pl.ds(0, 128)], in_v.at[m], ld_sem
        ).wait()

    def store_start(i, m):
        pltpu.async_copy(
            slab_v.at[m], packed_hbm.at[pl.ds(unit_of(i) * 64, 64)], st_sem
        )

    def store_wait(m):
        pltpu.make_async_copy(
            slab_v.at[m], packed_hbm.at[pl.ds(0, 64)], st_sem
        ).wait()

    def transpose_unit(m):
        src = in_v.at[m]
        dst = slab_v.at[m]
        iota = _iota16()
        two_iota = iota + iota
        perms = [lax.bitwise_and(iota + j, 15) for j in range(16)]

        def do_cb(cb, carry):
            half = lax.shift_right_logical(cb, 2)
            cbm16 = lax.shift_left(lax.bitwise_and(cb, 3), 4)
            cb16 = lax.shift_left(cb, 4)
            for pb in range(4):
                drows = iota + (16 * pb)
                sbase = two_iota + half + (32 * pb)
                for jg in range(2):
                    vals = []
                    for j in range(8 * jg, 8 * jg + 8):
                        srows = perms[j] + cbm16
                        vals.append(plsc.load_gather(src, [srows, sbase]))
                    for i, j in enumerate(range(8 * jg, 8 * jg + 8)):
                        dcols = perms[j] + cb16
                        plsc.store_scatter(dst, [drows, dcols], vals[i])
            return carry

        lax.fori_loop(0, 8, do_cb, 0)

    load_start(0, 0)

    @pl.when(n_units > 1)
    def _():
        load_start(1, 1)

    def body(i, carry):
        m = i % 2
        load_wait(m)

        @pl.when(i >= 2)
        def _():
            store_wait(m)

        transpose_unit(m)
        store_start(i, m)

        @pl.when(i + 2 < n_units)
        def _():
            load_start(i + 2, m)

        return carry

    lax.fori_loop(0, n_units, body, 0)

    @pl.when(n_units >= 2)
    def _():
        store_wait((n_units - 2) % 2)

    store_wait((n_units - 1) % 2)

    @pl.when(wid == 0)
    def _():
        pltpu.sync_copy(tail_hbm, in_v.at[0, pl.ds(0, 32)])
        pltpu.sync_copy(
            in_v.at[0, pl.ds(0, 32)],
            packed_hbm.at[pl.ds(_UNITS * 64, 32)],
        )


@functools.partial(
    pl.kernel,
    mesh=_mesh,
    out_type=jax.ShapeDtypeStruct((_S, _DIM, _B), jnp.float32),
    scratch_types=[
        pltpu.VMEM((_S, 128), jnp.int32),
        pltpu.VMEM((_S, 128), jnp.int32),
        pltpu.VMEM((2, 128, 128), jnp.float32),
        pltpu.VMEM((2, _DIM, 128), jnp.float32),
        pltpu.SemaphoreType.DMA,
        pltpu.SemaphoreType.DMA,
    ],
    compiler_params=pltpu.CompilerParams(use_tc_tiling_on_sc=True, needs_layout_passes=False),
)
def _lookup(ids_t_hbm, packed_hbm, out_t_hbm, idx_v, sel_v, buf_v, slab_v,
            gat_sem, st_sem):
    wid = lax.axis_index("s") * _NC + lax.axis_index("c")
    bcol = wid * 128

    pltpu.sync_copy(ids_t_hbm.at[:, pl.ds(bcol, 128)], idx_v)

    def prep_row(r, carry):
        for k in range(8):
            v = idx_v[r, pl.ds(16 * k, 16)]
            idx_v[r, pl.ds(16 * k, 16)] = lax.shift_right_logical(v, 1)
            sel_v[r, pl.ds(16 * k, 16)] = lax.shift_left(
                lax.bitwise_and(v, 1), 6
            )
        return carry

    lax.fori_loop(0, _S, prep_row, 0)

    def gather_start(s, m):
        pltpu.async_copy(packed_hbm.at[idx_v.at[s]], buf_v.at[m], gat_sem)

    def gather_wait(s, m):
        pltpu.make_async_copy(
            packed_hbm.at[idx_v.at[s]], buf_v.at[m], gat_sem
        ).wait()

    def store_start(s, m):
        pltpu.async_copy(
            slab_v.at[m], out_t_hbm.at[s, :, pl.ds(bcol, 128)], st_sem
        )

    def store_wait(m):
        pltpu.make_async_copy(
            slab_v.at[m], out_t_hbm.at[0, :, pl.ds(bcol, 128)], st_sem
        ).wait()

    def transpose_tokens(s, bm, sm):
        src = buf_v.at[bm]
        dst = slab_v.at[sm]
        iota = _iota16()
        srow = jnp.full((16,), 0, jnp.int32) + s
        perms = [lax.bitwise_and(iota + j, 15) for j in range(16)]

        def do_jb(jb, carry):
            toks = iota + lax.shift_left(jb, 4)
            selv = plsc.load_gather(sel_v, [srow, toks])
            for j in range(16):
                for dg in range(2):
                    dvecs = [perms[j] + (16 * db) for db in range(2 * dg, 2 * dg + 2)]
                    vals = [plsc.load_gather(src, [toks, selv + dv]) for dv in dvecs]
                    for dv, val in zip(dvecs, vals):
                        plsc.store_scatter(dst, [dv, toks], val)
            return carry

        lax.fori_loop(0, 8, do_jb, 0)

    gather_start(0, 0)
    gather_start(1, 1)

    def body(s, carry):
        bm = s % 2
        sm = s % 2
        gather_wait(s, bm)

        @pl.when(s >= 2)
        def _():
            store_wait(sm)

        transpose_tokens(s, bm, sm)
        store_start(s, sm)

        @pl.when(s + 2 < _S)
        def _():
            gather_start(s + 2, bm)

        return carry

    lax.fori_loop(0, _S, body, 0)

    store_wait(0)
    store_wait(1)


def kernel(input_ids, tok_emb):
    tail = tok_emb[_UNITS * 128:].reshape(32, 128)
    packed = _repack(tok_emb.T, tail)
    out_t = _lookup(input_ids.T, packed)
    return out_t.transpose(2, 0, 1)
